# baseline (speedup 1.0000x reference)
import functools
import math

import jax
import jax.numpy as jnp
from jax.experimental import pallas as pl
from jax.experimental.pallas import tpu as pltpu

_PI = math.pi

_ACOS4_PI = tuple(
    c / _PI for c in (-0.0187293, 0.0742610, -0.2121144, 1.5707288))
_INV_PI = 1.0 / _PI


def _acospi_poly(x):
    p = jnp.float32(_ACOS4_PI[0])
    for c in _ACOS4_PI[1:]:
        p = p * x + jnp.float32(c)
    return p


def _ag_kernel(a_ref, g_ref, b_ref, ab_ref):
    a = a_ref[...]
    b_ref[...] = jax.lax.dot_general(
        a, g_ref[...], (((1,), (0,)), ((), ())),
        preferred_element_type=jnp.float32)
    ab_ref[...] = a.astype(ab_ref.dtype)


def _theta_kernel(b1_ref, b2_ref, o_ref, agg_ref, row_ref):
    bm = b1_ref.shape[0]
    b1 = b1_ref[...]
    b2 = b2_ref[...]
    agg_ref[...] = jax.lax.dot_general(
        b1, b2, (((1,), (1,)), ((), ())),
        preferred_element_type=jnp.float32)
    n1 = jnp.sum(b1 * b1, axis=1, keepdims=True)
    r1 = jax.lax.rsqrt(n1)
    row_ref[:, 0:1] = r1
    row_ref[:, 1:2] = n1 * r1
    n2 = jnp.sum(b2 * b2, axis=1, keepdims=True).T
    r2t = jax.lax.rsqrt(n2)
    d2t = n2 * r2t

    def body(i, carry):
        sl = pl.ds(i * 8, 8)
        a = agg_ref[sl, :]
        r1c = row_ref[sl, 0:1]
        d1c = row_ref[sl, 1:2]
        s = jnp.clip((a * r1c) * r2t, -0.9999, 0.9999)
        ax = jnp.abs(s)
        t = 1.0 - ax
        rp = jnp.sqrt(t) * _acospi_poly(ax)
        w = jnp.where(s >= 0, 1.0 - rp, rp)
        sq1p = jnp.sqrt(t * (1.0 + ax)) * jnp.float32(_INV_PI)
        k1 = s * w + sq1p
        degree = 1.0 - jnp.sqrt(1.0 - k1) * _acospi_poly(k1)
        o_ref[sl, :] = (a * degree + (k1 * d1c) * d2t).astype(o_ref.dtype)
        return carry

    jax.lax.fori_loop(0, bm // 8, body, 0)


def _mm_kernel(x_ref, y_ref, o_ref, *, trans_y):
    dn = (((1,), (1 if trans_y else 0,)), ((), ()))
    o_ref[...] = jax.lax.dot_general(
        x_ref[...], y_ref[...], dn,
        preferred_element_type=jnp.float32).astype(o_ref.dtype)


def _matmul(x, y, trans_y, bm, bn, out_dtype):
    M, K = x.shape
    N = y.shape[0] if trans_y else y.shape[1]
    if trans_y:
        y_spec = pl.BlockSpec((bn, K), lambda m, n: (n, 0))
    else:
        y_spec = pl.BlockSpec((K, bn), lambda m, n: (0, n))
    return pl.pallas_call(
        functools.partial(_mm_kernel, trans_y=trans_y),
        grid=(M // bm, N // bn),
        in_specs=[pl.BlockSpec((bm, K), lambda m, n: (m, 0)), y_spec],
        out_specs=pl.BlockSpec((bm, bn), lambda m, n: (m, n)),
        out_shape=jax.ShapeDtypeStruct((M, N), out_dtype),
        compiler_params=pltpu.CompilerParams(
            dimension_semantics=("parallel", "parallel")),
    )(x, y)


def _a_times_g(A, g, bm):
    M, K = A.shape
    D = g.shape[1]
    return pl.pallas_call(
        _ag_kernel,
        grid=(M // bm,),
        in_specs=[
            pl.BlockSpec((bm, K), lambda m: (m, 0)),
            pl.BlockSpec((K, D), lambda m: (0, 0)),
        ],
        out_specs=[
            pl.BlockSpec((bm, D), lambda m: (m, 0)),
            pl.BlockSpec((bm, K), lambda m: (m, 0)),
        ],
        out_shape=[
            jax.ShapeDtypeStruct((M, D), jnp.float32),
            jax.ShapeDtypeStruct((M, K), jnp.bfloat16),
        ],
        compiler_params=pltpu.CompilerParams(
            dimension_semantics=("parallel",)),
    )(A, g)


def _theta(B1, B2, bm, bn, out_dtype):
    M = B1.shape[0]
    N = B2.shape[0]
    D = B1.shape[1]
    return pl.pallas_call(
        _theta_kernel,
        grid=(M // bm, N // bn),
        in_specs=[
            pl.BlockSpec((bm, D), lambda m, n: (m, 0)),
            pl.BlockSpec((bn, D), lambda m, n: (n, 0)),
        ],
        out_specs=pl.BlockSpec((bm, bn), lambda m, n: (m, n)),
        out_shape=jax.ShapeDtypeStruct((M, N), out_dtype),
        scratch_shapes=[
            pltpu.VMEM((bm, bn), jnp.float32),
            pltpu.VMEM((bm, 8), jnp.float32),
        ],
        compiler_params=pltpu.CompilerParams(
            dimension_semantics=("parallel", "parallel")),
    )(B1, B2)


def kernel(g1, g2, A1, A2):
    B1, A1b = _a_times_g(A1, g1, bm=512)
    B2, A2b = _a_times_g(A2, g2, bm=512)
    theta = _theta(B1, B2, bm=512, bn=512, out_dtype=jnp.bfloat16)
    T = _matmul(A1b, theta, trans_y=False, bm=1024, bn=1024,
                out_dtype=jnp.bfloat16)
    out = _matmul(T, A2b, trans_y=True, bm=1024, bn=1024,
                  out_dtype=jnp.float32)
    return out


# speedup vs baseline: 1.5031x; 1.5031x over previous
import functools
import math

import jax
import jax.numpy as jnp
from jax.experimental import pallas as pl
from jax.experimental.pallas import tpu as pltpu

_PI = math.pi

_ACOS4_PI = tuple(
    c / _PI for c in (-0.0187293, 0.0742610, -0.2121144, 1.5707288))
_INV_PI = 1.0 / _PI


def _acospi_poly(x):
    p = jnp.float32(_ACOS4_PI[0])
    for c in _ACOS4_PI[1:]:
        p = p * x + jnp.float32(c)
    return p


def _ag_kernel(a_ref, g_ref, b_ref, ab_ref):
    a = a_ref[...]
    b_ref[...] = jax.lax.dot_general(
        a, g_ref[...], (((1,), (0,)), ((), ())),
        preferred_element_type=jnp.float32)
    ab_ref[...] = a.astype(ab_ref.dtype)


def _theta_kernel(b1_ref, b2_ref, o_ref):
    b1 = b1_ref[...]
    b2 = b2_ref[...]
    agg = jax.lax.dot_general(
        b1, b2, (((1,), (1,)), ((), ())),
        preferred_element_type=jnp.float32)
    n1 = jnp.sum(b1 * b1, axis=1, keepdims=True)
    r1 = jax.lax.rsqrt(n1)
    d1 = n1 * r1
    n2 = jnp.sum(b2 * b2, axis=1, keepdims=True).T
    r2t = jax.lax.rsqrt(n2)
    d2t = n2 * r2t
    s = jnp.clip((agg * r1) * r2t, -0.9999, 0.9999)
    ax = jnp.abs(s)
    t = 1.0 - ax
    rp = jnp.sqrt(t) * _acospi_poly(ax)
    w = jnp.where(s >= 0, 1.0 - rp, rp)
    sq1p = jnp.sqrt(t * (1.0 + ax)) * jnp.float32(_INV_PI)
    k1 = s * w + sq1p
    degree = 1.0 - jnp.sqrt(1.0 - k1) * _acospi_poly(k1)
    o_ref[...] = (agg * degree + (k1 * d1) * d2t).astype(o_ref.dtype)


def _mm_kernel(x_ref, y_ref, o_ref, *, trans_y):
    dn = (((1,), (1 if trans_y else 0,)), ((), ()))
    o_ref[...] = jax.lax.dot_general(
        x_ref[...], y_ref[...], dn,
        preferred_element_type=jnp.float32).astype(o_ref.dtype)


def _matmul(x, y, trans_y, bm, bn, out_dtype):
    M, K = x.shape
    N = y.shape[0] if trans_y else y.shape[1]
    if trans_y:
        y_spec = pl.BlockSpec((bn, K), lambda m, n: (n, 0))
    else:
        y_spec = pl.BlockSpec((K, bn), lambda m, n: (0, n))
    return pl.pallas_call(
        functools.partial(_mm_kernel, trans_y=trans_y),
        grid=(M // bm, N // bn),
        in_specs=[pl.BlockSpec((bm, K), lambda m, n: (m, 0)), y_spec],
        out_specs=pl.BlockSpec((bm, bn), lambda m, n: (m, n)),
        out_shape=jax.ShapeDtypeStruct((M, N), out_dtype),
        compiler_params=pltpu.CompilerParams(
            dimension_semantics=("parallel", "parallel")),
    )(x, y)


def _a_times_g(A, g, bm):
    M, K = A.shape
    D = g.shape[1]
    return pl.pallas_call(
        _ag_kernel,
        grid=(M // bm,),
        in_specs=[
            pl.BlockSpec((bm, K), lambda m: (m, 0)),
            pl.BlockSpec((K, D), lambda m: (0, 0)),
        ],
        out_specs=[
            pl.BlockSpec((bm, D), lambda m: (m, 0)),
            pl.BlockSpec((bm, K), lambda m: (m, 0)),
        ],
        out_shape=[
            jax.ShapeDtypeStruct((M, D), jnp.float32),
            jax.ShapeDtypeStruct((M, K), jnp.bfloat16),
        ],
        compiler_params=pltpu.CompilerParams(
            dimension_semantics=("parallel",)),
    )(A, g)


def _theta(B1, B2, bm, bn, out_dtype):
    M = B1.shape[0]
    N = B2.shape[0]
    D = B1.shape[1]
    return pl.pallas_call(
        _theta_kernel,
        grid=(M // bm, N // bn),
        in_specs=[
            pl.BlockSpec((bm, D), lambda m, n: (m, 0)),
            pl.BlockSpec((bn, D), lambda m, n: (n, 0)),
        ],
        out_specs=pl.BlockSpec((bm, bn), lambda m, n: (m, n)),
        out_shape=jax.ShapeDtypeStruct((M, N), out_dtype),
        compiler_params=pltpu.CompilerParams(
            dimension_semantics=("parallel", "parallel")),
    )(B1, B2)


def kernel(g1, g2, A1, A2):
    B1, A1b = _a_times_g(A1, g1, bm=512)
    B2, A2b = _a_times_g(A2, g2, bm=512)
    theta = _theta(B1, B2, bm=512, bn=512, out_dtype=jnp.bfloat16)
    T = _matmul(A1b, theta, trans_y=False, bm=1024, bn=1024,
                out_dtype=jnp.bfloat16)
    out = _matmul(T, A2b, trans_y=True, bm=1024, bn=1024,
                  out_dtype=jnp.float32)
    return out


# speedup vs baseline: 1.6719x; 1.1123x over previous
import functools
import math

import jax
import jax.numpy as jnp
from jax.experimental import pallas as pl
from jax.experimental.pallas import tpu as pltpu

_PI = math.pi

_ACOS4_PI = tuple(
    c / _PI for c in (-0.0187293, 0.0742610, -0.2121144, 1.5707288))
_INV_PI = 1.0 / _PI


def _acospi_poly(x):
    p = jnp.float32(_ACOS4_PI[0])
    for c in _ACOS4_PI[1:]:
        p = p * x + jnp.float32(c)
    return p


def _ag_kernel(a_ref, g_ref, b_ref, ab_ref):
    a = a_ref[...]
    b_ref[...] = jax.lax.dot_general(
        a, g_ref[...], (((1,), (0,)), ((), ())),
        preferred_element_type=jnp.float32)
    ab_ref[...] = a.astype(ab_ref.dtype)


def _theta_kernel(b1_ref, b2_ref, o_ref):
    b1 = b1_ref[...]
    b2 = b2_ref[...]
    agg = jax.lax.dot_general(
        b1, b2, (((1,), (1,)), ((), ())),
        preferred_element_type=jnp.float32)
    n1 = jnp.sum(b1 * b1, axis=1, keepdims=True)
    r1 = jax.lax.rsqrt(n1)
    d1 = n1 * r1
    n2 = jnp.sum(b2 * b2, axis=1, keepdims=True).T
    r2t = jax.lax.rsqrt(n2)
    d2t = n2 * r2t
    s = jnp.clip((agg * r1) * r2t, -0.9999, 0.9999)
    ax = jnp.abs(s)
    t = 1.0 - ax
    rp = (t * jax.lax.rsqrt(t)) * _acospi_poly(ax)
    w = jnp.where(s >= 0, 1.0 - rp, rp)
    u = t * (1.0 + ax)
    sq1p = (u * jax.lax.rsqrt(u)) * jnp.float32(_INV_PI)
    k1 = s * w + sq1p
    t2 = 1.0 - k1
    degree = 1.0 - (t2 * jax.lax.rsqrt(t2)) * _acospi_poly(k1)
    o_ref[...] = (agg * degree + (k1 * d1) * d2t).astype(o_ref.dtype)


def _mm_kernel(x_ref, y_ref, o_ref, *, trans_y):
    dn = (((1,), (1 if trans_y else 0,)), ((), ()))
    o_ref[...] = jax.lax.dot_general(
        x_ref[...], y_ref[...], dn,
        preferred_element_type=jnp.float32).astype(o_ref.dtype)


def _matmul(x, y, trans_y, bm, bn, out_dtype):
    M, K = x.shape
    N = y.shape[0] if trans_y else y.shape[1]
    if trans_y:
        y_spec = pl.BlockSpec((bn, K), lambda m, n: (n, 0))
    else:
        y_spec = pl.BlockSpec((K, bn), lambda m, n: (0, n))
    return pl.pallas_call(
        functools.partial(_mm_kernel, trans_y=trans_y),
        grid=(M // bm, N // bn),
        in_specs=[pl.BlockSpec((bm, K), lambda m, n: (m, 0)), y_spec],
        out_specs=pl.BlockSpec((bm, bn), lambda m, n: (m, n)),
        out_shape=jax.ShapeDtypeStruct((M, N), out_dtype),
        compiler_params=pltpu.CompilerParams(
            dimension_semantics=("parallel", "parallel")),
    )(x, y)


def _a_times_g(A, g, bm):
    M, K = A.shape
    D = g.shape[1]
    return pl.pallas_call(
        _ag_kernel,
        grid=(M // bm,),
        in_specs=[
            pl.BlockSpec((bm, K), lambda m: (m, 0)),
            pl.BlockSpec((K, D), lambda m: (0, 0)),
        ],
        out_specs=[
            pl.BlockSpec((bm, D), lambda m: (m, 0)),
            pl.BlockSpec((bm, K), lambda m: (m, 0)),
        ],
        out_shape=[
            jax.ShapeDtypeStruct((M, D), jnp.float32),
            jax.ShapeDtypeStruct((M, K), jnp.bfloat16),
        ],
        compiler_params=pltpu.CompilerParams(
            dimension_semantics=("parallel",)),
    )(A, g)


def _theta(B1, B2, bm, bn, out_dtype):
    M = B1.shape[0]
    N = B2.shape[0]
    D = B1.shape[1]
    return pl.pallas_call(
        _theta_kernel,
        grid=(M // bm, N // bn),
        in_specs=[
            pl.BlockSpec((bm, D), lambda m, n: (m, 0)),
            pl.BlockSpec((bn, D), lambda m, n: (n, 0)),
        ],
        out_specs=pl.BlockSpec((bm, bn), lambda m, n: (m, n)),
        out_shape=jax.ShapeDtypeStruct((M, N), out_dtype),
        compiler_params=pltpu.CompilerParams(
            dimension_semantics=("parallel", "parallel")),
    )(B1, B2)


def kernel(g1, g2, A1, A2):
    B1, A1b = _a_times_g(A1, g1, bm=512)
    B2, A2b = _a_times_g(A2, g2, bm=512)
    theta = _theta(B1, B2, bm=512, bn=512, out_dtype=jnp.bfloat16)
    T = _matmul(A1b, theta, trans_y=False, bm=1024, bn=1024,
                out_dtype=jnp.bfloat16)
    out = _matmul(T, A2b, trans_y=True, bm=1024, bn=1024,
                  out_dtype=jnp.float32)
    return out
